# Initial kernel scaffold; baseline (speedup 1.0000x reference)
import jax
import jax.numpy as jnp
from jax import lax
from jax.experimental import pallas as pl

NUM_GRAPHS = 512
D_FEAT = 128
N_NODES = 100000

BLOCK = 2500
NB = N_NODES // BLOCK


def _gather_block(x_ref, idx_ref, out_ref):
    idx = idx_ref[0, 0, :]
    onehot = (idx[:, None] == lax.broadcasted_iota(jnp.int32, (BLOCK, NUM_GRAPHS), 1)
              ).astype(jnp.float32)
    out_ref[...] = jnp.dot(onehot, x_ref[...], preferred_element_type=jnp.float32)


def kernel(X, batch):
    idx = batch.astype(jnp.int32).reshape(NB, 1, BLOCK)
    out = pl.pallas_call(
        _gather_block,
        grid=(NB,),
        in_specs=[
            pl.BlockSpec((NUM_GRAPHS, D_FEAT), lambda i: (0, 0)),
            pl.BlockSpec((1, 1, BLOCK), lambda i: (i, 0, 0)),
        ],
        out_specs=pl.BlockSpec((BLOCK, D_FEAT), lambda i: (i, 0)),
        out_shape=jax.ShapeDtypeStruct((N_NODES, D_FEAT), jnp.float32),
    )(X, idx)
    return out


# speedup vs baseline: 4.9020x; 4.9020x over previous
import jax
import jax.numpy as jnp
from jax import lax
from jax.experimental import pallas as pl

NUM_GRAPHS = 512
D_FEAT = 128
N_NODES = 100000

BLOCK = 2000
NB = N_NODES // BLOCK


def _gather_block(x_ref, idx_ref, out_ref):
    idx = idx_ref[0, 0, :]
    onehot = (idx[:, None] == lax.broadcasted_iota(jnp.int32, (BLOCK, NUM_GRAPHS), 1)
              ).astype(jnp.float32)
    out_ref[...] = jnp.dot(onehot, x_ref[...], preferred_element_type=jnp.float32)


def kernel(X, batch):
    idx = batch.astype(jnp.int32).reshape(NB, 1, BLOCK)
    out = pl.pallas_call(
        _gather_block,
        grid=(NB,),
        in_specs=[
            pl.BlockSpec((NUM_GRAPHS, D_FEAT), lambda i: (0, 0)),
            pl.BlockSpec((1, 1, BLOCK), lambda i: (i, 0, 0)),
        ],
        out_specs=pl.BlockSpec((BLOCK, D_FEAT), lambda i: (i, 0)),
        out_shape=jax.ShapeDtypeStruct((N_NODES, D_FEAT), jnp.float32),
    )(X, idx)
    return out
